# baseline (speedup 1.0000x reference)
import numpy as np
import jax
import jax.numpy as jnp
from jax.experimental import pallas as pl
from jax.experimental.pallas import tpu as pltpu

_SEED = 1234
_D = 1024
_ROT = ((13, 15, 26, 6), (17, 29, 16, 24))


def _np_tf2x32(k1, k2, x1, x2):
    m = 0xFFFFFFFF

    def rotl(x, r):
        return ((x << r) | (x >> (32 - r))) & m

    ks = (k1 & m, k2 & m, (k1 ^ k2 ^ 0x1BD11BDA) & m)
    x0 = (x1 + ks[0]) & m
    xx = (x2 + ks[1]) & m
    for g in range(5):
        for r in _ROT[g % 2]:
            x0 = (x0 + xx) & m
            xx = (rotl(xx, r) ^ x0) & m
        x0 = (x0 + ks[(g + 1) % 3]) & m
        xx = (xx + ks[(g + 2) % 3] + g + 1) & m
    return x0, xx


def _np_split(key):
    a0, b0 = _np_tf2x32(key[0], key[1], 0, 0)
    a1, b1 = _np_tf2x32(key[0], key[1], 0, 1)
    return (a0, b0), (a1, b1)


_K_INTS, _K_SHUF = _np_split((0, _SEED))
_R1, _R2 = _np_split(_K_INTS)


def _i32c(v):
    v &= 0xFFFFFFFF
    return jnp.int32(v - (1 << 32) if v >= (1 << 31) else v)


def _rotl(x, r):
    return jax.lax.shift_left(x, jnp.int32(r)) | jax.lax.shift_right_logical(
        x, jnp.int32(32 - r)
    )


def _tf_bits(key, p):
    k1, k2 = key
    ks = (k1, k2, k1 ^ k2 ^ 0x1BD11BDA)
    x0 = jnp.full(p.shape, 0, jnp.int32) + _i32c(ks[0])
    x1 = p + _i32c(ks[1])
    for g in range(5):
        for r in _ROT[g % 2]:
            x0 = x0 + x1
            x1 = _rotl(x1, r) ^ x0
        x0 = x0 + _i32c(ks[(g + 1) % 3])
        x1 = x1 + _i32c(ks[(g + 2) % 3] + g + 1)
    return x0 ^ x1


def _body(x_ref, o_ref):
    R = x_ref.shape[0]
    D = _D
    b = pl.program_id(0)
    col = jax.lax.broadcasted_iota(jnp.int32, (R, D), 1)
    row = jax.lax.broadcasted_iota(jnp.int32, (R, D), 0) + b * R
    bits = _tf_bits(_K_SHUF, row * D + col)

    prow = jax.lax.broadcasted_iota(jnp.int32, (R, 1), 0) + b * R
    ints = _tf_bits(_R2, prow) & 1023

    K = (bits & _i32c(0xFFFFFE00)) | jax.lax.shift_right_logical(col, 1)
    F = K ^ _i32c(0x80000000)

    bits_at_i = jnp.sum(jnp.where(col == ints, bits, 0), axis=1, keepdims=True)
    key_bad_f = (
        (bits_at_i & _i32c(0xFFFFFE00))
        | jax.lax.shift_right_logical(ints, 1)
    ) ^ _i32c(0x80000000)

    neg_prev = jnp.zeros((R, D), jnp.int32)
    for s in range(1, 11):
        neg = -((col >> s) & 1) if s < 10 else jnp.zeros((R, D), jnp.int32)
        F = F ^ (neg ^ neg_prev)
        neg_prev = neg
        for t in range(s - 1, -1, -1):
            d = 1 << t
            lower = (col & d) == 0
            bb = pltpu.roll(F, D - d, 1)
            mn = jnp.minimum(F, bb)
            mx = jnp.maximum(F, bb)
            F = jnp.where(lower, mn, pltpu.roll(mx, d, 1))

    m2 = (F & 511) << 1
    g = jnp.where((m2 < ints) & (F != key_bad_f), 1.0, 0.0).astype(x_ref.dtype)
    o_ref[:, :D] = x_ref[:] * g
    o_ref[:, D:] = g


def kernel(x):
    B, D = x.shape
    R = 8
    return pl.pallas_call(
        _body,
        out_shape=jax.ShapeDtypeStruct((B, 2 * D), x.dtype),
        grid=(B // R,),
        in_specs=[pl.BlockSpec((R, D), lambda b: (b, 0))],
        out_specs=pl.BlockSpec((R, 2 * D), lambda b: (b, 0)),
        compiler_params=pltpu.CompilerParams(
            dimension_semantics=("arbitrary",),
        ),
    )(x)


# speedup vs baseline: 5.2275x; 5.2275x over previous
import numpy as np
import jax
import jax.numpy as jnp
from jax.experimental import pallas as pl
from jax.experimental.pallas import tpu as pltpu

_SEED = 1234
_D = 1024
_ROT = ((13, 15, 26, 6), (17, 29, 16, 24))


def _np_tf2x32(k1, k2, x1, x2):
    m = 0xFFFFFFFF

    def rotl(x, r):
        return ((x << r) | (x >> (32 - r))) & m

    ks = (k1 & m, k2 & m, (k1 ^ k2 ^ 0x1BD11BDA) & m)
    x0 = (x1 + ks[0]) & m
    xx = (x2 + ks[1]) & m
    for g in range(5):
        for r in _ROT[g % 2]:
            x0 = (x0 + xx) & m
            xx = (rotl(xx, r) ^ x0) & m
        x0 = (x0 + ks[(g + 1) % 3]) & m
        xx = (xx + ks[(g + 2) % 3] + g + 1) & m
    return x0, xx


def _np_split(key):
    a0, b0 = _np_tf2x32(key[0], key[1], 0, 0)
    a1, b1 = _np_tf2x32(key[0], key[1], 0, 1)
    return (a0, b0), (a1, b1)


_K_INTS, _K_SHUF = _np_split((0, _SEED))
_R1, _R2 = _np_split(_K_INTS)


def _i32c(v):
    v &= 0xFFFFFFFF
    return jnp.int32(v - (1 << 32) if v >= (1 << 31) else v)


def _rotl(x, r):
    return jax.lax.shift_left(x, jnp.int32(r)) | jax.lax.shift_right_logical(
        x, jnp.int32(32 - r)
    )


def _tf_bits(key, p):
    k1, k2 = key
    ks = (k1, k2, k1 ^ k2 ^ 0x1BD11BDA)
    x0 = jnp.full(p.shape, 0, jnp.int32) + _i32c(ks[0])
    x1 = p + _i32c(ks[1])
    for g in range(5):
        for r in _ROT[g % 2]:
            x0 = x0 + x1
            x1 = _rotl(x1, r) ^ x0
        x0 = x0 + _i32c(ks[(g + 1) % 3])
        x1 = x1 + _i32c(ks[(g + 2) % 3] + g + 1)
    return x0 ^ x1


def _body(x_ref, o_ref):
    R = x_ref.shape[0]
    D = _D
    b = pl.program_id(0)
    col = jax.lax.broadcasted_iota(jnp.int32, (R, D), 1)
    row = jax.lax.broadcasted_iota(jnp.int32, (R, D), 0) + b * R
    bits = _tf_bits(_K_SHUF, row * D + col)

    prow = jax.lax.broadcasted_iota(jnp.int32, (R, 1), 0) + b * R
    ints = _tf_bits(_R2, prow) & 1023

    K = (bits & _i32c(0xFFFFFE00)) | jax.lax.shift_right_logical(col, 1)
    F = K ^ _i32c(0x80000000)

    bits_at_i = jnp.sum(jnp.where(col == ints, bits, 0), axis=1, keepdims=True)
    key_bad_f = (
        (bits_at_i & _i32c(0xFFFFFE00))
        | jax.lax.shift_right_logical(ints, 1)
    ) ^ _i32c(0x80000000)

    neg_prev = jnp.zeros((R, D), jnp.int32)
    for s in range(1, 11):
        neg = -((col >> s) & 1) if s < 10 else jnp.zeros((R, D), jnp.int32)
        F = F ^ (neg ^ neg_prev)
        neg_prev = neg
        for t in range(s - 1, -1, -1):
            d = 1 << t
            lower = (col & d) == 0
            bb = pltpu.roll(F, D - d, 1)
            mn = jnp.minimum(F, bb)
            mx = jnp.maximum(F, bb)
            F = jnp.where(lower, mn, pltpu.roll(mx, d, 1))

    m2 = (F & 511) << 1
    g = jnp.where((m2 < ints) & (F != key_bad_f), 1.0, 0.0).astype(x_ref.dtype)
    o_ref[:, :D] = x_ref[:] * g
    o_ref[:, D:] = g


def kernel(x):
    B, D = x.shape
    R = 64
    return pl.pallas_call(
        _body,
        out_shape=jax.ShapeDtypeStruct((B, 2 * D), x.dtype),
        grid=(B // R,),
        in_specs=[pl.BlockSpec((R, D), lambda b: (b, 0))],
        out_specs=pl.BlockSpec((R, 2 * D), lambda b: (b, 0)),
        compiler_params=pltpu.CompilerParams(
            dimension_semantics=("arbitrary",),
        ),
    )(x)


# speedup vs baseline: 6.2543x; 1.1964x over previous
import numpy as np
import jax
import jax.numpy as jnp
from jax.experimental import pallas as pl
from jax.experimental.pallas import tpu as pltpu

_SEED = 1234
_D = 1024
_ROT = ((13, 15, 26, 6), (17, 29, 16, 24))


def _np_tf2x32(k1, k2, x1, x2):
    m = 0xFFFFFFFF

    def rotl(x, r):
        return ((x << r) | (x >> (32 - r))) & m

    ks = (k1 & m, k2 & m, (k1 ^ k2 ^ 0x1BD11BDA) & m)
    x0 = (x1 + ks[0]) & m
    xx = (x2 + ks[1]) & m
    for g in range(5):
        for r in _ROT[g % 2]:
            x0 = (x0 + xx) & m
            xx = (rotl(xx, r) ^ x0) & m
        x0 = (x0 + ks[(g + 1) % 3]) & m
        xx = (xx + ks[(g + 2) % 3] + g + 1) & m
    return x0, xx


def _np_split(key):
    a0, b0 = _np_tf2x32(key[0], key[1], 0, 0)
    a1, b1 = _np_tf2x32(key[0], key[1], 0, 1)
    return (a0, b0), (a1, b1)


_K_INTS, _K_SHUF = _np_split((0, _SEED))
_R1, _R2 = _np_split(_K_INTS)


def _i32c(v):
    v &= 0xFFFFFFFF
    return jnp.int32(v - (1 << 32) if v >= (1 << 31) else v)


def _rotl(x, r):
    return jax.lax.shift_left(x, jnp.int32(r)) | jax.lax.shift_right_logical(
        x, jnp.int32(32 - r)
    )


def _tf_bits(key, p):
    k1, k2 = key
    ks = (k1, k2, k1 ^ k2 ^ 0x1BD11BDA)
    x0 = jnp.full(p.shape, 0, jnp.int32) + _i32c(ks[0])
    x1 = p + _i32c(ks[1])
    for g in range(5):
        for r in _ROT[g % 2]:
            x0 = x0 + x1
            x1 = _rotl(x1, r) ^ x0
        x0 = x0 + _i32c(ks[(g + 1) % 3])
        x1 = x1 + _i32c(ks[(g + 2) % 3] + g + 1)
    return x0 ^ x1


def _body(x_ref, o_ref):
    R = x_ref.shape[0]
    D = _D
    b = pl.program_id(0)
    col = jax.lax.broadcasted_iota(jnp.int32, (R, D), 1)
    row = jax.lax.broadcasted_iota(jnp.int32, (R, D), 0) + b * R
    bits = _tf_bits(_K_SHUF, row * D + col)

    prow = jax.lax.broadcasted_iota(jnp.int32, (R, 1), 0) + b * R
    ints = _tf_bits(_R2, prow) & 1023

    K = (bits & _i32c(0xFFFFFE00)) | jax.lax.shift_right_logical(col, 1)
    F = K ^ _i32c(0x80000000)

    bits_at_i = jnp.sum(jnp.where(col == ints, bits, 0), axis=1, keepdims=True)
    key_bad_f = (
        (bits_at_i & _i32c(0xFFFFFE00))
        | jax.lax.shift_right_logical(ints, 1)
    ) ^ _i32c(0x80000000)

    neg_prev = jnp.zeros((R, D), jnp.int32)
    for s in range(1, 11):
        neg = -((col >> s) & 1) if s < 10 else jnp.zeros((R, D), jnp.int32)
        F = F ^ (neg ^ neg_prev)
        neg_prev = neg
        for t in range(s - 1, -1, -1):
            d = 1 << t
            lower = (col & d) == 0
            bb = pltpu.roll(F, D - d, 1)
            mn = jnp.minimum(F, bb)
            mx = jnp.maximum(F, bb)
            F = jnp.where(lower, mn, pltpu.roll(mx, d, 1))

    m2 = (F & 511) << 1
    g = jnp.where((m2 < ints) & (F != key_bad_f), 1.0, 0.0).astype(x_ref.dtype)
    o_ref[:, :D] = x_ref[:] * g
    o_ref[:, D:] = g


def kernel(x):
    B, D = x.shape
    R = 512
    return pl.pallas_call(
        _body,
        out_shape=jax.ShapeDtypeStruct((B, 2 * D), x.dtype),
        grid=(B // R,),
        in_specs=[pl.BlockSpec((R, D), lambda b: (b, 0))],
        out_specs=pl.BlockSpec((R, 2 * D), lambda b: (b, 0)),
        compiler_params=pltpu.CompilerParams(
            dimension_semantics=("arbitrary",),
        ),
    )(x)


# speedup vs baseline: 6.4565x; 1.0323x over previous
import numpy as np
import jax
import jax.numpy as jnp
from jax.experimental import pallas as pl
from jax.experimental.pallas import tpu as pltpu

_SEED = 1234
_D = 1024
_ROT = ((13, 15, 26, 6), (17, 29, 16, 24))


def _np_tf2x32(k1, k2, x1, x2):
    m = 0xFFFFFFFF

    def rotl(x, r):
        return ((x << r) | (x >> (32 - r))) & m

    ks = (k1 & m, k2 & m, (k1 ^ k2 ^ 0x1BD11BDA) & m)
    x0 = (x1 + ks[0]) & m
    xx = (x2 + ks[1]) & m
    for g in range(5):
        for r in _ROT[g % 2]:
            x0 = (x0 + xx) & m
            xx = (rotl(xx, r) ^ x0) & m
        x0 = (x0 + ks[(g + 1) % 3]) & m
        xx = (xx + ks[(g + 2) % 3] + g + 1) & m
    return x0, xx


def _np_split(key):
    a0, b0 = _np_tf2x32(key[0], key[1], 0, 0)
    a1, b1 = _np_tf2x32(key[0], key[1], 0, 1)
    return (a0, b0), (a1, b1)


_K_INTS, _K_SHUF = _np_split((0, _SEED))
_R1, _R2 = _np_split(_K_INTS)


def _i32c(v):
    v &= 0xFFFFFFFF
    return jnp.int32(v - (1 << 32) if v >= (1 << 31) else v)


def _rotl(x, r):
    return jax.lax.shift_left(x, jnp.int32(r)) | jax.lax.shift_right_logical(
        x, jnp.int32(32 - r)
    )


def _tf_bits(key, p):
    k1, k2 = key
    ks = (k1, k2, k1 ^ k2 ^ 0x1BD11BDA)
    x0 = jnp.full(p.shape, 0, jnp.int32) + _i32c(ks[0])
    x1 = p + _i32c(ks[1])
    for g in range(5):
        for r in _ROT[g % 2]:
            x0 = x0 + x1
            x1 = _rotl(x1, r) ^ x0
        x0 = x0 + _i32c(ks[(g + 1) % 3])
        x1 = x1 + _i32c(ks[(g + 2) % 3] + g + 1)
    return x0 ^ x1


def _body(x_ref, o_ref):
    R = x_ref.shape[0]
    D = _D
    b = pl.program_id(0)
    col = jax.lax.broadcasted_iota(jnp.int32, (R, D), 1)
    row = jax.lax.broadcasted_iota(jnp.int32, (R, D), 0) + b * R
    bits = _tf_bits(_K_SHUF, row * D + col)

    prow = jax.lax.broadcasted_iota(jnp.int32, (R, 1), 0) + b * R
    ints = _tf_bits(_R2, prow) & 1023

    K = (bits & _i32c(0xFFFFFE00)) | jax.lax.shift_right_logical(col, 1)
    F = K ^ _i32c(0x80000000)

    bits_at_i = jnp.sum(jnp.where(col == ints, bits, 0), axis=1, keepdims=True)
    key_bad_f = (
        (bits_at_i & _i32c(0xFFFFFE00))
        | jax.lax.shift_right_logical(ints, 1)
    ) ^ _i32c(0x80000000)

    neg_prev = jnp.zeros((R, D), jnp.int32)
    for s in range(1, 11):
        neg = -((col >> s) & 1) if s < 10 else jnp.zeros((R, D), jnp.int32)
        F = F ^ (neg ^ neg_prev)
        neg_prev = neg
        for t in range(s - 1, -1, -1):
            d = 1 << t
            lower = (col & d) == 0
            bb = pltpu.roll(F, D - d, 1)
            mn = jnp.minimum(F, bb)
            mx = jnp.maximum(F, bb)
            F = jnp.where(lower, mn, pltpu.roll(mx, d, 1))

    m2 = (F & 511) << 1
    g = jnp.where((m2 < ints) & (F != key_bad_f), 1.0, 0.0).astype(x_ref.dtype)
    o_ref[:, :D] = x_ref[:] * g
    o_ref[:, D:] = g


def kernel(x):
    B, D = x.shape
    R = 128
    return pl.pallas_call(
        _body,
        out_shape=jax.ShapeDtypeStruct((B, 2 * D), x.dtype),
        grid=(B // R,),
        in_specs=[pl.BlockSpec((R, D), lambda b: (b, 0))],
        out_specs=pl.BlockSpec((R, 2 * D), lambda b: (b, 0)),
        compiler_params=pltpu.CompilerParams(
            dimension_semantics=("arbitrary",),
        ),
    )(x)


# speedup vs baseline: 7.0915x; 1.0983x over previous
import numpy as np
import jax
import jax.numpy as jnp
from jax.experimental import pallas as pl
from jax.experimental.pallas import tpu as pltpu

_SEED = 1234
_D = 1024
_ROT = ((13, 15, 26, 6), (17, 29, 16, 24))


def _np_tf2x32(k1, k2, x1, x2):
    m = 0xFFFFFFFF

    def rotl(x, r):
        return ((x << r) | (x >> (32 - r))) & m

    ks = (k1 & m, k2 & m, (k1 ^ k2 ^ 0x1BD11BDA) & m)
    x0 = (x1 + ks[0]) & m
    xx = (x2 + ks[1]) & m
    for g in range(5):
        for r in _ROT[g % 2]:
            x0 = (x0 + xx) & m
            xx = (rotl(xx, r) ^ x0) & m
        x0 = (x0 + ks[(g + 1) % 3]) & m
        xx = (xx + ks[(g + 2) % 3] + g + 1) & m
    return x0, xx


def _np_split(key):
    a0, b0 = _np_tf2x32(key[0], key[1], 0, 0)
    a1, b1 = _np_tf2x32(key[0], key[1], 0, 1)
    return (a0, b0), (a1, b1)


_K_INTS, _K_SHUF = _np_split((0, _SEED))
_R1, _R2 = _np_split(_K_INTS)


def _i32c(v):
    v &= 0xFFFFFFFF
    return jnp.int32(v - (1 << 32) if v >= (1 << 31) else v)


def _rotl(x, r):
    return jax.lax.shift_left(x, jnp.int32(r)) | jax.lax.shift_right_logical(
        x, jnp.int32(32 - r)
    )


def _tf_bits(key, p):
    k1, k2 = key
    ks = (k1, k2, k1 ^ k2 ^ 0x1BD11BDA)
    x0 = jnp.full(p.shape, 0, jnp.int32) + _i32c(ks[0])
    x1 = p + _i32c(ks[1])
    for g in range(5):
        for r in _ROT[g % 2]:
            x0 = x0 + x1
            x1 = _rotl(x1, r) ^ x0
        x0 = x0 + _i32c(ks[(g + 1) % 3])
        x1 = x1 + _i32c(ks[(g + 2) % 3] + g + 1)
    return x0 ^ x1


def _body(x_ref, o_ref):
    R = x_ref.shape[0]
    D = _D
    b = pl.program_id(0)
    col = jax.lax.broadcasted_iota(jnp.int32, (R, D), 1)
    row = jax.lax.broadcasted_iota(jnp.int32, (R, D), 0) + b * R
    jlog = ((col & 127) << 3) | jax.lax.shift_right_logical(col, 7)
    bits = _tf_bits(_K_SHUF, row * D + jlog)

    prow = jax.lax.broadcasted_iota(jnp.int32, (R, 1), 0) + b * R
    ints = _tf_bits(_R2, prow) & 1023

    K = (bits & _i32c(0xFFFFFE00)) | jax.lax.shift_right_logical(jlog, 1)
    F = K ^ _i32c(0x80000000)

    bits_at_i = jnp.sum(jnp.where(jlog == ints, bits, 0), axis=1, keepdims=True)
    key_bad_f = (
        (bits_at_i & _i32c(0xFFFFFE00))
        | jax.lax.shift_right_logical(ints, 1)
    ) ^ _i32c(0x80000000)

    neg_prev = jnp.zeros((R, D), jnp.int32)
    for s in range(1, 11):
        neg = -((jlog >> s) & 1) if s < 10 else jnp.zeros((R, D), jnp.int32)
        F = F ^ (neg ^ neg_prev)
        neg_prev = neg
        for t in range(s - 1, -1, -1):
            d = 1 << t
            pd = 128 << t if t < 3 else 1 << (t - 3)
            lower = (jlog & d) == 0
            bb = pltpu.roll(F, D - pd, 1)
            mn = jnp.minimum(F, bb)
            mx = jnp.maximum(F, bb)
            F = jnp.where(lower, mn, pltpu.roll(mx, pd, 1))

    m2 = (F & 511) << 1
    g = jnp.where((m2 < ints) & (F != key_bad_f), 1.0, 0.0).astype(x_ref.dtype)
    g = jnp.reshape(jnp.swapaxes(jnp.reshape(g, (R, 8, 128)), 1, 2), (R, D))
    o_ref[:, :D] = x_ref[:] * g
    o_ref[:, D:] = g


def kernel(x):
    B, D = x.shape
    R = 256
    return pl.pallas_call(
        _body,
        out_shape=jax.ShapeDtypeStruct((B, 2 * D), x.dtype),
        grid=(B // R,),
        in_specs=[pl.BlockSpec((R, D), lambda b: (b, 0))],
        out_specs=pl.BlockSpec((R, 2 * D), lambda b: (b, 0)),
        compiler_params=pltpu.CompilerParams(
            dimension_semantics=("arbitrary",),
        ),
    )(x)
